# baseline (speedup 1.0000x reference)
import jax
import jax.numpy as jnp
from jax.experimental import pallas as pl
from jax.experimental.pallas import tpu as pltpu

IN_DIMS = 1024
HID = 1024
OUT = 1024
B = 4096

_R = 512
_NB = B // _R


def _body(x_ref, ws_ref, bs_ref, g0_ref, b0_ref, we_ref, be_ref,
          out_ref, h_s, acc_s, b2_s):
    p = pl.program_id(0)
    i = pl.program_id(1)

    @pl.when(p == 0)
    def _phase0():
        h = jnp.dot(x_ref[:], ws_ref[:],
                    preferred_element_type=jnp.float32)
        h = jnp.maximum(h + bs_ref[:], 0.0)
        h_s[pl.ds(i * _R, _R), :] = h
        colsum = jnp.sum(h, axis=0, keepdims=True)
        colsq = jnp.sum(h * h, axis=0, keepdims=True)

        @pl.when(i == 0)
        def _init():
            acc_s[0:1, :] = colsum
            acc_s[1:2, :] = colsq

        @pl.when(i > 0)
        def _accum():
            acc_s[0:1, :] = acc_s[0:1, :] + colsum
            acc_s[1:2, :] = acc_s[1:2, :] + colsq

    @pl.when(p == 1)
    def _phase1():
        @pl.when(i == 0)
        def _bn_params():
            m = acc_s[0:1, :] * (1.0 / B)
            v = acc_s[1:2, :] * (1.0 / B) - m * m
            s = g0_ref[:] * jax.lax.rsqrt(v + 1e-5)
            b2_s[0:1, :] = s
            b2_s[1:2, :] = b0_ref[:] - m * s

        hb = h_s[pl.ds(i * _R, _R), :] * b2_s[0:1, :] + b2_s[1:2, :]
        o = jnp.dot(hb, we_ref[:], preferred_element_type=jnp.float32)
        out_ref[:] = jnp.maximum(o + be_ref[:], 0.0)


def kernel(x, W_start, b_start, bn0_g, bn0_b, W_exp, b_exp, bn_g, bn_b,
           W_end, b_end, W_dqn, b_dqn):
    del W_exp, b_exp, bn_g, bn_b, W_dqn, b_dqn

    row = lambda a: a.reshape(1, -1)
    grid = (2, _NB)
    out = pl.pallas_call(
        _body,
        grid=grid,
        in_specs=[
            pl.BlockSpec((_R, IN_DIMS), lambda p, i: (i * (1 - p), 0)),
            pl.BlockSpec((IN_DIMS, HID), lambda p, i: (0, 0)),
            pl.BlockSpec((1, HID), lambda p, i: (0, 0)),
            pl.BlockSpec((1, HID), lambda p, i: (0, 0)),
            pl.BlockSpec((1, HID), lambda p, i: (0, 0)),
            pl.BlockSpec((HID, OUT), lambda p, i: (0, 0)),
            pl.BlockSpec((1, OUT), lambda p, i: (0, 0)),
        ],
        out_specs=pl.BlockSpec((_R, OUT), lambda p, i: (i * p, 0)),
        out_shape=jax.ShapeDtypeStruct((B, OUT), jnp.float32),
        scratch_shapes=[
            pltpu.VMEM((B, HID), jnp.float32),
            pltpu.VMEM((2, HID), jnp.float32),
            pltpu.VMEM((2, HID), jnp.float32),
        ],
        compiler_params=pltpu.CompilerParams(
            dimension_semantics=("arbitrary", "arbitrary"),
        ),
    )(x, W_start, row(b_start), row(bn0_g), row(bn0_b), W_end, row(b_end))
    return out


# speedup vs baseline: 1.1023x; 1.1023x over previous
import jax
import jax.numpy as jnp
from jax.experimental import pallas as pl
from jax.experimental.pallas import tpu as pltpu

IN_DIMS = 1024
HID = 1024
OUT = 1024
B = 4096

_R = 1024
_NB = B // _R


def _body(x_ref, ws_ref, bs_ref, g0_ref, b0_ref, we_ref, be_ref,
          out_ref, h_s, acc_s, b2_s, we_s, we_sem):
    p = pl.program_id(0)
    i = pl.program_id(1)

    @pl.when(p == 0)
    def _phase0():
        @pl.when(i == 0)
        def _start_we_copy():
            pltpu.make_async_copy(we_ref, we_s, we_sem).start()

        h = jnp.dot(x_ref[:], ws_ref[:],
                    preferred_element_type=jnp.float32)
        h = jnp.maximum(h + bs_ref[:], 0.0)
        h_s[pl.ds(i * _R, _R), :] = h
        colsum = jnp.sum(h, axis=0, keepdims=True)
        colsq = jnp.sum(h * h, axis=0, keepdims=True)

        @pl.when(i == 0)
        def _init():
            acc_s[0:1, :] = colsum
            acc_s[1:2, :] = colsq

        @pl.when(i > 0)
        def _accum():
            acc_s[0:1, :] = acc_s[0:1, :] + colsum
            acc_s[1:2, :] = acc_s[1:2, :] + colsq

    @pl.when(p == 1)
    def _phase1():
        @pl.when(i == 0)
        def _bn_params():
            pltpu.make_async_copy(we_ref, we_s, we_sem).wait()
            m = acc_s[0:1, :] * (1.0 / B)
            v = acc_s[1:2, :] * (1.0 / B) - m * m
            s = g0_ref[:] * jax.lax.rsqrt(v + 1e-5)
            b2_s[0:1, :] = s
            b2_s[1:2, :] = b0_ref[:] - m * s

        hb = h_s[pl.ds(i * _R, _R), :] * b2_s[0:1, :] + b2_s[1:2, :]
        o = jnp.dot(hb, we_s[:, :], preferred_element_type=jnp.float32)
        out_ref[:] = jnp.maximum(o + be_ref[:], 0.0)


def kernel(x, W_start, b_start, bn0_g, bn0_b, W_exp, b_exp, bn_g, bn_b,
           W_end, b_end, W_dqn, b_dqn):
    del W_exp, b_exp, bn_g, bn_b, W_dqn, b_dqn

    row = lambda a: a.reshape(1, -1)
    grid = (2, _NB)
    out = pl.pallas_call(
        _body,
        grid=grid,
        in_specs=[
            pl.BlockSpec((_R, IN_DIMS), lambda p, i: (i * (1 - p), 0)),
            pl.BlockSpec((IN_DIMS, HID), lambda p, i: (0, 0)),
            pl.BlockSpec((1, HID), lambda p, i: (0, 0)),
            pl.BlockSpec((1, HID), lambda p, i: (0, 0)),
            pl.BlockSpec((1, HID), lambda p, i: (0, 0)),
            pl.BlockSpec(memory_space=pl.ANY),
            pl.BlockSpec((1, OUT), lambda p, i: (0, 0)),
        ],
        out_specs=pl.BlockSpec((_R, OUT), lambda p, i: (i * p, 0)),
        out_shape=jax.ShapeDtypeStruct((B, OUT), jnp.float32),
        scratch_shapes=[
            pltpu.VMEM((B, HID), jnp.float32),
            pltpu.VMEM((2, HID), jnp.float32),
            pltpu.VMEM((2, HID), jnp.float32),
            pltpu.VMEM((HID, OUT), jnp.float32),
            pltpu.SemaphoreType.DMA,
        ],
        compiler_params=pltpu.CompilerParams(
            dimension_semantics=("arbitrary", "arbitrary"),
        ),
    )(x, W_start, row(b_start), row(bn0_g), row(bn0_b), W_end, row(b_end))
    return out
